# baseline (speedup 1.0000x reference)
import functools

import jax
import jax.numpy as jnp
from jax import lax
from jax.experimental import pallas as pl
from jax.experimental.pallas import tpu as pltpu
from jax.experimental.pallas import tpu_sc as plsc

_QT = 256
_NT = 512
_KPAD = 8


def _score_topk_kernel(topk, hw, qt, x_ref, vals_ref, idx_ref, xn_ref):
    b = pl.program_id(0)
    qi = pl.program_id(1)

    @pl.when(qi == 0)
    def _():
        xv = x_ref[0]
        nrm = jnp.sqrt(jnp.sum(xv * xv, axis=0, keepdims=True))
        xn_ref[...] = xv / nrm

    xn = xn_ref[...]
    xq = xn_ref[:, pl.ds(qi * qt, qt)]
    s = lax.dot_general(
        xq, xn, (((0,), (0,)), ((), ())),
        preferred_element_type=jnp.float32,
        precision=lax.Precision.DEFAULT,
    )

    iota = lax.broadcasted_iota(jnp.int32, (qt, hw), 1).astype(jnp.float32)
    neg = jnp.float32(jnp.finfo(jnp.float32).min)
    big = jnp.float32(2.0 ** 30)
    m = jnp.max(s, axis=1)
    for t in range(topk):
        cand = jnp.where(s == m[:, None], iota, big)
        am = jnp.min(cand, axis=1)
        vals_ref[0, t, :] = m
        idx_ref[0, t, :] = am.astype(jnp.int32) + b * hw
        if t + 1 < topk:
            s = jnp.where(iota == am[:, None], neg, s)
            m = jnp.max(s, axis=1)
    for t in range(topk, _KPAD):
        vals_ref[0, t, :] = jnp.zeros((qt,), jnp.float32)
        idx_ref[0, t, :] = jnp.zeros((qt,), jnp.int32)


def _score_topk(x2, topk):
    b, c, hw = x2.shape
    qt = _QT
    grid = (b, hw // qt)
    out_shapes = [
        jax.ShapeDtypeStruct((b, _KPAD, hw), jnp.float32),
        jax.ShapeDtypeStruct((b, _KPAD, hw), jnp.int32),
    ]
    return pl.pallas_call(
        functools.partial(_score_topk_kernel, topk, hw, qt),
        grid=grid,
        in_specs=[pl.BlockSpec((1, c, hw), lambda bi, qi: (bi, 0, 0))],
        out_specs=[
            pl.BlockSpec((1, _KPAD, qt), lambda bi, qi: (bi, 0, qi)),
            pl.BlockSpec((1, _KPAD, qt), lambda bi, qi: (bi, 0, qi)),
        ],
        out_shape=out_shapes,
        scratch_shapes=[pltpu.VMEM((c, hw), jnp.float32)],
    )(x2)


def _sc_gather(table, idx2d, nch):
    nrows, c = table.shape
    nw = 32
    nchp = idx2d.shape[0] // nw
    ck = 128
    rpw = nch * ck
    mesh = plsc.VectorSubcoreMesh(core_axis_name="c", subcore_axis_name="s")

    @functools.partial(
        pl.kernel,
        mesh=mesh,
        out_type=jax.ShapeDtypeStruct((nw * rpw, c), jnp.float32),
        scratch_types=[
            pltpu.VMEM((nchp, ck), jnp.int32),
            pltpu.VMEM((ck, c), jnp.float32),
            pltpu.VMEM((ck, c), jnp.float32),
            pltpu.SemaphoreType.DMA,
            pltpu.SemaphoreType.DMA,
        ],
    )
    def k(table_hbm, idx_hbm, out_hbm, idx_v, buf0, buf1, sem0, sem1):
        wid = lax.axis_index("s") * 2 + lax.axis_index("c")
        base = wid * rpw
        pltpu.sync_copy(idx_hbm.at[pl.ds(wid * nchp, nchp)], idx_v)
        bufs = (buf0, buf1)
        sems = (sem0, sem1)
        handles = [None, None]
        handles[0] = pltpu.async_copy(table_hbm.at[idx_v.at[0]], bufs[0], sems[0])
        for j in range(nch):
            if j + 1 < nch:
                handles[(j + 1) % 2] = pltpu.async_copy(
                    table_hbm.at[idx_v.at[j + 1]], bufs[(j + 1) % 2],
                    sems[(j + 1) % 2])
            handles[j % 2].wait()
            pltpu.sync_copy(bufs[j % 2], out_hbm.at[pl.ds(base + j * ck, ck)])

    return k(table, idx2d)


def _combine_kernel(topk, nt, g_ref, v_ref, w_ref, b_ref, o_ref):
    acc = jnp.zeros((w_ref.shape[1], nt), jnp.float32)
    for t in range(topk):
        gs = g_ref[0, t] * v_ref[0, t, :][:, None]
        acc = acc + lax.dot_general(
            w_ref[t], gs, (((1,), (1,)), ((), ())),
            preferred_element_type=jnp.float32,
            precision=lax.Precision.DEFAULT,
        )
    o_ref[0] = jnp.maximum(acc + b_ref[...], 0.0)


def _combine(g4, vals, wkt, b2d):
    b, _, hw, cp = g4.shape
    topk, c, _ = wkt.shape
    nt = _NT
    grid = (b, hw // nt)
    return pl.pallas_call(
        functools.partial(_combine_kernel, topk, nt),
        grid=grid,
        in_specs=[
            pl.BlockSpec((1, topk, nt, cp), lambda bi, ni: (bi, 0, ni, 0)),
            pl.BlockSpec((1, _KPAD, nt), lambda bi, ni: (bi, 0, ni)),
            pl.BlockSpec((topk, c, cp), lambda bi, ni: (0, 0, 0)),
            pl.BlockSpec((c, 1), lambda bi, ni: (0, 0)),
        ],
        out_specs=pl.BlockSpec((1, c, nt), lambda bi, ni: (bi, 0, ni)),
        out_shape=jax.ShapeDtypeStruct((b, c, hw), jnp.float32),
    )(g4, vals, wkt, b2d)


def kernel(x, W, b_conv):
    b, c, h, w = x.shape
    hw = h * w
    topk = W.shape[2]
    cp = 128
    x2 = x.reshape(b, c, hw)
    wkt = jnp.transpose(W[:, :, :, 0, 0], (2, 0, 1))
    wkt = jnp.pad(wkt, ((0, 0), (0, 0), (0, cp - c)))
    b2d = b_conv.reshape(c, 1)
    nw = 32
    nch = (topk * hw) // (nw * 128)
    nchp = (nch + 7) // 8 * 8
    outs = []
    for bi in range(b):
        xb = lax.slice_in_dim(x2, bi, bi + 1, axis=0)
        table = jnp.pad(xb[0].T, ((0, 0), (0, cp - c)))
        vals, idxg = _score_topk(xb, topk)
        idx3 = idxg[:, :topk, :].reshape(nw, nch, 128)
        idx3 = jnp.pad(idx3, ((0, 0), (0, nchp - nch), (0, 0)))
        g = _sc_gather(table, idx3.reshape(nw * nchp, 128), nch)
        g4 = g.reshape(1, topk, hw, cp)
        outs.append(_combine(g4, vals, wkt, b2d))
    return jnp.concatenate(outs, axis=0).reshape(b, c, h, w)


# speedup vs baseline: 1.0098x; 1.0098x over previous
import functools

import jax
import jax.numpy as jnp
from jax import lax
from jax.experimental import pallas as pl
from jax.experimental.pallas import tpu as pltpu
from jax.experimental.pallas import tpu_sc as plsc

_QT = 256
_NT = 2048
_KPAD = 8


def _score_topk_kernel(topk, hw, qt, x_ref, vals_ref, idx_ref, xn_ref):
    b = pl.program_id(0)
    qi = pl.program_id(1)

    @pl.when(qi == 0)
    def _():
        xv = x_ref[0]
        nrm = jnp.sqrt(jnp.sum(xv * xv, axis=0, keepdims=True))
        xn_ref[...] = xv / nrm

    xn = xn_ref[...]
    xq = xn_ref[:, pl.ds(qi * qt, qt)]
    s = lax.dot_general(
        xq, xn, (((0,), (0,)), ((), ())),
        preferred_element_type=jnp.float32,
        precision=lax.Precision.DEFAULT,
    )

    iota = lax.broadcasted_iota(jnp.int32, (qt, hw), 1).astype(jnp.float32)
    neg = jnp.float32(jnp.finfo(jnp.float32).min)
    big = jnp.float32(2.0 ** 30)
    m = jnp.max(s, axis=1)
    for t in range(topk):
        cand = jnp.where(s == m[:, None], iota, big)
        am = jnp.min(cand, axis=1)
        vals_ref[0, t, :] = m
        idx_ref[0, t, :] = am.astype(jnp.int32) + b * hw
        if t + 1 < topk:
            s = jnp.where(iota == am[:, None], neg, s)
            m = jnp.max(s, axis=1)
    for t in range(topk, _KPAD):
        vals_ref[0, t, :] = jnp.zeros((qt,), jnp.float32)
        idx_ref[0, t, :] = jnp.zeros((qt,), jnp.int32)


def _score_topk(x2, topk):
    b, c, hw = x2.shape
    qt = _QT
    grid = (b, hw // qt)
    out_shapes = [
        jax.ShapeDtypeStruct((b, _KPAD, hw), jnp.float32),
        jax.ShapeDtypeStruct((b, _KPAD, hw), jnp.int32),
    ]
    return pl.pallas_call(
        functools.partial(_score_topk_kernel, topk, hw, qt),
        grid=grid,
        in_specs=[pl.BlockSpec((1, c, hw), lambda bi, qi: (bi, 0, 0))],
        out_specs=[
            pl.BlockSpec((1, _KPAD, qt), lambda bi, qi: (bi, 0, qi)),
            pl.BlockSpec((1, _KPAD, qt), lambda bi, qi: (bi, 0, qi)),
        ],
        out_shape=out_shapes,
        scratch_shapes=[pltpu.VMEM((c, hw), jnp.float32)],
    )(x2)


def _sc_gather(table, idx2d, nch):
    nrows, c = table.shape
    nw = 32
    nchp = idx2d.shape[0] // nw
    ck = 128
    rpw = nch * ck
    mesh = plsc.VectorSubcoreMesh(core_axis_name="c", subcore_axis_name="s")

    @functools.partial(
        pl.kernel,
        mesh=mesh,
        out_type=jax.ShapeDtypeStruct((nw * rpw, c), jnp.float32),
        scratch_types=[
            pltpu.VMEM((nchp, ck), jnp.int32),
            pltpu.VMEM((ck, c), jnp.float32),
            pltpu.VMEM((ck, c), jnp.float32),
            pltpu.SemaphoreType.DMA,
            pltpu.SemaphoreType.DMA,
        ],
    )
    def k(table_hbm, idx_hbm, out_hbm, idx_v, buf0, buf1, sem0, sem1):
        wid = lax.axis_index("s") * 2 + lax.axis_index("c")
        base = wid * rpw
        pltpu.sync_copy(idx_hbm.at[pl.ds(wid * nchp, nchp)], idx_v)
        bufs = (buf0, buf1)
        sems = (sem0, sem1)
        handles = [None, None]
        handles[0] = pltpu.async_copy(table_hbm.at[idx_v.at[0]], bufs[0], sems[0])
        for j in range(nch):
            if j + 1 < nch:
                handles[(j + 1) % 2] = pltpu.async_copy(
                    table_hbm.at[idx_v.at[j + 1]], bufs[(j + 1) % 2],
                    sems[(j + 1) % 2])
            handles[j % 2].wait()
            pltpu.sync_copy(bufs[j % 2], out_hbm.at[pl.ds(base + j * ck, ck)])

    return k(table, idx2d)


def _combine_kernel(topk, nt, g_ref, v_ref, w_ref, b_ref, o_ref):
    acc = jnp.zeros((w_ref.shape[1], nt), jnp.float32)
    for t in range(topk):
        gs = g_ref[0, t] * v_ref[0, t, :][:, None]
        acc = acc + lax.dot_general(
            w_ref[t], gs, (((1,), (1,)), ((), ())),
            preferred_element_type=jnp.float32,
            precision=lax.Precision.DEFAULT,
        )
    o_ref[0] = jnp.maximum(acc + b_ref[...], 0.0)


def _combine(g4, vals, wkt, b2d):
    b, _, hw, cp = g4.shape
    topk, c, _ = wkt.shape
    nt = _NT
    grid = (b, hw // nt)
    return pl.pallas_call(
        functools.partial(_combine_kernel, topk, nt),
        grid=grid,
        in_specs=[
            pl.BlockSpec((1, topk, nt, cp), lambda bi, ni: (bi, 0, ni, 0)),
            pl.BlockSpec((1, _KPAD, nt), lambda bi, ni: (bi, 0, ni)),
            pl.BlockSpec((topk, c, cp), lambda bi, ni: (0, 0, 0)),
            pl.BlockSpec((c, 1), lambda bi, ni: (0, 0)),
        ],
        out_specs=pl.BlockSpec((1, c, nt), lambda bi, ni: (bi, 0, ni)),
        out_shape=jax.ShapeDtypeStruct((b, c, hw), jnp.float32),
    )(g4, vals, wkt, b2d)


def kernel(x, W, b_conv):
    b, c, h, w = x.shape
    hw = h * w
    topk = W.shape[2]
    cp = 128
    x2 = x.reshape(b, c, hw)
    wkt = jnp.transpose(W[:, :, :, 0, 0], (2, 0, 1))
    wkt = jnp.pad(wkt, ((0, 0), (0, 0), (0, cp - c)))
    b2d = b_conv.reshape(c, 1)
    nw = 32
    nch = (topk * hw) // (nw * 128)
    nchp = (nch + 7) // 8 * 8
    outs = []
    for bi in range(b):
        xb = lax.slice_in_dim(x2, bi, bi + 1, axis=0)
        table = jnp.pad(xb[0].T, ((0, 0), (0, cp - c)))
        vals, idxg = _score_topk(xb, topk)
        idx3 = idxg[:, :topk, :].reshape(nw, nch, 128)
        idx3 = jnp.pad(idx3, ((0, 0), (0, nchp - nch), (0, 0)))
        g = _sc_gather(table, idx3.reshape(nw * nchp, 128), nch)
        g4 = g.reshape(1, topk, hw, cp)
        outs.append(_combine(g4, vals, wkt, b2d))
    return jnp.concatenate(outs, axis=0).reshape(b, c, h, w)


# speedup vs baseline: 1.0181x; 1.0082x over previous
import functools

import jax
import jax.numpy as jnp
from jax import lax
from jax.experimental import pallas as pl
from jax.experimental.pallas import tpu as pltpu
from jax.experimental.pallas import tpu_sc as plsc

_QT = 256
_NT = 2048
_KPAD = 8


def _score_topk_kernel(topk, hw, qt, x_ref, vals_ref, idx_ref, xn_ref):
    qi = pl.program_id(1)

    @pl.when(qi == 0)
    def _():
        xv = x_ref[0]
        nrm = jnp.sqrt(jnp.sum(xv * xv, axis=0, keepdims=True))
        xn_ref[...] = xv / nrm

    xn = xn_ref[...]
    xq = xn_ref[:, pl.ds(qi * qt, qt)]
    s = lax.dot_general(
        xq, xn, (((0,), (0,)), ((), ())),
        preferred_element_type=jnp.float32,
        precision=lax.Precision.DEFAULT,
    )

    iota = lax.broadcasted_iota(jnp.int32, (qt, hw), 1).astype(jnp.float32)
    neg = jnp.float32(jnp.finfo(jnp.float32).min)
    big = jnp.float32(2.0 ** 30)
    m = jnp.max(s, axis=1)
    for t in range(topk):
        cand = jnp.where(s == m[:, None], iota, big)
        am = jnp.min(cand, axis=1)
        vals_ref[0, t, :] = m
        idx_ref[0, t, :] = am
        if t + 1 < topk:
            s = jnp.where(iota == am[:, None], neg, s)
            m = jnp.max(s, axis=1)
    for t in range(topk, _KPAD):
        vals_ref[0, t, :] = jnp.zeros((qt,), jnp.float32)
        idx_ref[0, t, :] = jnp.zeros((qt,), jnp.float32)


def _score_topk(x2, topk):
    b, c, hw = x2.shape
    qt = _QT
    grid = (b, hw // qt)
    out_shapes = [
        jax.ShapeDtypeStruct((b, _KPAD, hw), jnp.float32),
        jax.ShapeDtypeStruct((b, _KPAD, hw), jnp.float32),
    ]
    return pl.pallas_call(
        functools.partial(_score_topk_kernel, topk, hw, qt),
        grid=grid,
        in_specs=[pl.BlockSpec((1, c, hw), lambda bi, qi: (bi, 0, 0))],
        out_specs=[
            pl.BlockSpec((1, _KPAD, qt), lambda bi, qi: (bi, 0, qi)),
            pl.BlockSpec((1, _KPAD, qt), lambda bi, qi: (bi, 0, qi)),
        ],
        out_shape=out_shapes,
        scratch_shapes=[pltpu.VMEM((c, hw), jnp.float32)],
    )(x2)


def _sc_gather(table, idx2d, nch):
    nrows, c = table.shape
    nw = 32
    nchp = idx2d.shape[0] // nw
    ck = 128
    rpw = nch * ck
    mesh = plsc.VectorSubcoreMesh(core_axis_name="c", subcore_axis_name="s")

    @functools.partial(
        pl.kernel,
        mesh=mesh,
        out_type=jax.ShapeDtypeStruct((nw * rpw, c), jnp.float32),
        scratch_types=[
            pltpu.VMEM((nchp, ck), jnp.int32),
            pltpu.VMEM((ck, c), jnp.float32),
            pltpu.VMEM((ck, c), jnp.float32),
            pltpu.SemaphoreType.DMA,
            pltpu.SemaphoreType.DMA,
        ],
    )
    def k(table_hbm, idx_hbm, out_hbm, idx_v, buf0, buf1, sem0, sem1):
        wid = lax.axis_index("s") * 2 + lax.axis_index("c")
        base = wid * rpw
        pltpu.sync_copy(idx_hbm.at[pl.ds(wid * nchp, nchp)], idx_v)
        bufs = (buf0, buf1)
        sems = (sem0, sem1)
        handles = [None, None]
        handles[0] = pltpu.async_copy(table_hbm.at[idx_v.at[0]], bufs[0], sems[0])
        for j in range(nch):
            if j + 1 < nch:
                handles[(j + 1) % 2] = pltpu.async_copy(
                    table_hbm.at[idx_v.at[j + 1]], bufs[(j + 1) % 2],
                    sems[(j + 1) % 2])
            handles[j % 2].wait()
            pltpu.sync_copy(bufs[j % 2], out_hbm.at[pl.ds(base + j * ck, ck)])

    return k(table, idx2d)


def _combine_kernel(topk, nt, g_ref, v_ref, w_ref, b_ref, o_ref):
    acc = jnp.zeros((w_ref.shape[1], nt), jnp.float32)
    for t in range(topk):
        gs = g_ref[0, t] * v_ref[0, t, :][:, None]
        acc = acc + lax.dot_general(
            w_ref[t], gs, (((1,), (1,)), ((), ())),
            preferred_element_type=jnp.float32,
            precision=lax.Precision.DEFAULT,
        )
    o_ref[0] = jnp.maximum(acc + b_ref[...], 0.0)


def _combine(g4, vals, wkt, b2d):
    b, _, hw, cp = g4.shape
    topk, c, _ = wkt.shape
    nt = _NT
    grid = (b, hw // nt)
    return pl.pallas_call(
        functools.partial(_combine_kernel, topk, nt),
        grid=grid,
        in_specs=[
            pl.BlockSpec((1, topk, nt, cp), lambda bi, ni: (bi, 0, ni, 0)),
            pl.BlockSpec((1, _KPAD, nt), lambda bi, ni: (bi, 0, ni)),
            pl.BlockSpec((topk, c, cp), lambda bi, ni: (0, 0, 0)),
            pl.BlockSpec((c, 1), lambda bi, ni: (0, 0)),
        ],
        out_specs=pl.BlockSpec((1, c, nt), lambda bi, ni: (bi, 0, ni)),
        out_shape=jax.ShapeDtypeStruct((b, c, hw), jnp.float32),
    )(g4, vals, wkt, b2d)


def kernel(x, W, b_conv):
    b, c, h, w = x.shape
    hw = h * w
    topk = W.shape[2]
    cp = 128
    x2 = x.reshape(b, c, hw)
    wkt = jnp.transpose(W[:, :, :, 0, 0], (2, 0, 1))
    wkt = jnp.pad(wkt, ((0, 0), (0, 0), (0, cp - c)))
    b2d = b_conv.reshape(c, 1)
    nw = 32
    nch = (topk * hw) // (nw * 128)
    nchp = (nch + 7) // 8 * 8
    outs = []
    for bi in range(b):
        xb = lax.slice_in_dim(x2, bi, bi + 1, axis=0)
        table = jnp.pad(xb[0].T, ((0, 0), (0, cp - c)))
        vals, idxg = _score_topk(xb, topk)
        idx3 = idxg[:, :topk, :].astype(jnp.int32).reshape(nw, nch, 128)
        idx3 = jnp.pad(idx3, ((0, 0), (0, nchp - nch), (0, 0)))
        g = _sc_gather(table, idx3.reshape(nw * nchp, 128), nch)
        g4 = g.reshape(1, topk, hw, cp)
        outs.append(_combine(g4, vals, wkt, b2d))
    return jnp.concatenate(outs, axis=0).reshape(b, c, h, w)
